# Initial kernel scaffold; baseline (speedup 1.0000x reference)
import jax
import jax.numpy as jnp
from jax.experimental import pallas as pl


def kernel(z, embedding):
    raise NotImplementedError("write your pallas kernel here")


# speedup vs baseline: 1.6042x; 1.6042x over previous
import functools

import jax
import jax.numpy as jnp
from jax.experimental import pallas as pl


def _vq_kernel(z_ref, emb_ref, enc_ref, qflat_ref, idx_ref, quant_ref):
    zb = z_ref[0]
    emb = emb_ref[...]

    scores = jax.lax.dot_general(
        emb, zb, (((1,), (0,)), ((), ())),
        preferred_element_type=jnp.float32)
    emb_sq = jnp.sum(emb * emb, axis=1, keepdims=True)
    enc_sq = jnp.sum(zb * zb, axis=0, keepdims=True)
    dist = enc_sq - 2.0 * scores + emb_sq

    k_iota = jax.lax.broadcasted_iota(jnp.int32, dist.shape, 0)
    min_d = jnp.min(dist, axis=0, keepdims=True)
    idx = jnp.min(jnp.where(dist == min_d, k_iota, 1024), axis=0,
                  keepdims=True)
    idx_ref[0] = idx

    onehot = (k_iota == idx).astype(jnp.float32)
    quant = jax.lax.dot_general(
        emb, onehot, (((0,), (0,)), ((), ())),
        preferred_element_type=jnp.float32)
    quant_ref[0] = quant

    enc_ref[...] = zb.T
    qflat_ref[...] = quant.T


@jax.jit
def kernel(z, embedding):
    b, c, h, w = z.shape
    n = h * w
    k = embedding.shape[0]
    z3 = z.reshape(b, c, n)

    grid = (b,)
    enc_flat, q_flat, idx, quant = pl.pallas_call(
        _vq_kernel,
        grid=grid,
        in_specs=[
            pl.BlockSpec((1, c, n), lambda i: (i, 0, 0)),
            pl.BlockSpec((k, c), lambda i: (0, 0)),
        ],
        out_specs=[
            pl.BlockSpec((n, c), lambda i: (i, 0)),
            pl.BlockSpec((n, c), lambda i: (i, 0)),
            pl.BlockSpec((1, 1, n), lambda i: (i, 0, 0)),
            pl.BlockSpec((1, c, n), lambda i: (i, 0, 0)),
        ],
        out_shape=[
            jax.ShapeDtypeStruct((b * n, c), jnp.float32),
            jax.ShapeDtypeStruct((b * n, c), jnp.float32),
            jax.ShapeDtypeStruct((b, 1, n), jnp.int32),
            jax.ShapeDtypeStruct((b, c, n), jnp.float32),
        ],
    )(z3, embedding)

    codebook_indices = idx.reshape(b, h, w)
    quantized = quant.reshape(b, c, h, w)
    return (enc_flat, q_flat, codebook_indices, quantized)
